# Initial kernel scaffold; baseline (speedup 1.0000x reference)
import jax
import jax.numpy as jnp
from jax.experimental import pallas as pl


def kernel(seq1, seq2, h_3, edge_index, W_gcn, b_gcn, W_lin, b_lin):
    raise NotImplementedError("write your pallas kernel here")


# speedup vs baseline: 9.9903x; 9.9903x over previous
import functools

import jax
import jax.numpy as jnp
from jax import lax
from jax.experimental import pallas as pl
from jax.experimental.pallas import tpu as pltpu
from jax.experimental.pallas import tpu_sc as plsc

N = 10000
D = 256
E = 160000
NP = 10240
BLK = 1024
GRID = NP // BLK

NC = 2
NS = 16
CHUNK = 128
NCHUNK = 79
EPT_A = NCHUNK * CHUNK
E_PAD_A = NS * EPT_A
NCHUNK_D = 40
EPT_D = NCHUNK_D * CHUNK
E_PAD_D = NC * NS * EPT_D
DW = 16
ROWS_PT = NP // NS
RCHUNK = ROWS_PT // CHUNK

_MESH = plsc.VectorSubcoreMesh(
    core_axis_name="c", subcore_axis_name="s", num_cores=NC, num_subcores=NS)


@functools.partial(
    pl.kernel,
    out_type=[jax.ShapeDtypeStruct((NP, DW), jnp.float32)] * NC,
    mesh=_MESH,
    scratch_types=[
        pltpu.VMEM((NCHUNK_D, CHUNK), jnp.int32),
        pltpu.VMEM((CHUNK, DW), jnp.float32),
        pltpu.VMEM_SHARED((NP, DW), jnp.float32),
    ],
)
def _deg_kernel(dst_hbm, out0, out1, dst_v, buf, hist):
    c = lax.axis_index("c")
    s = lax.axis_index("s")
    wid = c * NS + s
    pltpu.sync_copy(dst_hbm.at[wid], dst_v)
    base = s * ROWS_PT

    def fill(val):
        def fbody(r, carry):
            buf[r] = jnp.full((DW,), val, jnp.float32)
            return carry
        lax.fori_loop(0, CHUNK, fbody, 0)

    def run(o_hbm):
        fill(0.0)

        def zbody(k, carry):
            pltpu.sync_copy(buf, hist.at[pl.ds(base + k * CHUNK, CHUNK)])
            return carry
        lax.fori_loop(0, RCHUNK, zbody, 0)
        plsc.subcore_barrier()

        fill(1.0)

        def ebody(j, carry):
            pltpu.sync_copy(buf, hist.at[dst_v.at[j]], add=True)
            return carry
        lax.fori_loop(0, NCHUNK_D, ebody, 0)
        plsc.subcore_barrier()

        def obody(k, carry):
            rows = pl.ds(base + k * CHUNK, CHUNK)
            pltpu.sync_copy(hist.at[rows], buf)
            pltpu.sync_copy(buf, o_hbm.at[rows])
            return carry
        lax.fori_loop(0, RCHUNK, obody, 0)

    @pl.when(c == 0)
    def _():
        run(out0)

    @pl.when(c == 1)
    def _():
        run(out1)


def _dense_body(seq1_ref, seq2_ref, deg0_ref, deg1_ref, w_ref,
                t0_ref, t1_ref, t2_ref, t3_ref, dinv_ref):
    deg = deg0_ref[...][:, 0] + deg1_ref[...][:, 0] + 1.0
    dinv = lax.rsqrt(deg)
    w = w_ref[...]
    g1 = jnp.dot(seq1_ref[...], w, preferred_element_type=jnp.float32)
    g1 = g1 * dinv[:, None]
    g2 = jnp.dot(seq2_ref[...], w, preferred_element_type=jnp.float32)
    g2 = g2 * dinv[:, None]
    t0_ref[...] = g1[:, :128]
    t1_ref[...] = g1[:, 128:]
    t2_ref[...] = g2[:, :128]
    t3_ref[...] = g2[:, 128:]
    dinv_ref[...] = dinv


_tab = jax.ShapeDtypeStruct((NP, 128), jnp.float32)
_dense_call = pl.pallas_call(
    _dense_body,
    grid=(GRID,),
    in_specs=[
        pl.BlockSpec((BLK, D), lambda i: (i, 0)),
        pl.BlockSpec((BLK, D), lambda i: (i, 0)),
        pl.BlockSpec((BLK, DW), lambda i: (i, 0)),
        pl.BlockSpec((BLK, DW), lambda i: (i, 0)),
        pl.BlockSpec((D, D), lambda i: (0, 0)),
    ],
    out_specs=[
        pl.BlockSpec((BLK, 128), lambda i: (i, 0)),
        pl.BlockSpec((BLK, 128), lambda i: (i, 0)),
        pl.BlockSpec((BLK, 128), lambda i: (i, 0)),
        pl.BlockSpec((BLK, 128), lambda i: (i, 0)),
        pl.BlockSpec((BLK,), lambda i: (i,)),
    ],
    out_shape=[_tab, _tab, _tab, _tab,
               jax.ShapeDtypeStruct((NP,), jnp.float32)],
)


@functools.partial(
    pl.kernel,
    out_type=[_tab, _tab, _tab, _tab],
    mesh=_MESH,
    scratch_types=[
        pltpu.VMEM((NCHUNK, CHUNK), jnp.int32),
        pltpu.VMEM((NCHUNK, CHUNK), jnp.int32),
        pltpu.VMEM((CHUNK, 128), jnp.float32),
        pltpu.VMEM_SHARED((NP, 128), jnp.float32),
        pltpu.SemaphoreType.DMA,
    ],
)
def _agg_kernel(t0, t1, t2, t3, src_hbm, dst_hbm,
                o0, o1, o2, o3, src_v, dst_v, buf, acc, sem):
    c = lax.axis_index("c")
    s = lax.axis_index("s")
    pltpu.sync_copy(src_hbm.at[s], src_v)
    pltpu.sync_copy(dst_hbm.at[s], dst_v)
    base = s * ROWS_PT

    def run_pass(t_hbm, o_hbm):
        def ibody(k, carry):
            rows = pl.ds(base + k * CHUNK, CHUNK)
            pltpu.sync_copy(t_hbm.at[rows], buf)
            pltpu.sync_copy(buf, acc.at[rows])
            return carry
        lax.fori_loop(0, RCHUNK, ibody, 0)
        plsc.subcore_barrier()

        def ebody(j, carry):
            pltpu.async_copy(t_hbm.at[src_v.at[j]], buf, sem).wait()
            pltpu.sync_copy(buf, acc.at[dst_v.at[j]], add=True)
            return carry
        lax.fori_loop(0, NCHUNK, ebody, 0)
        plsc.subcore_barrier()

        def obody(k, carry):
            rows = pl.ds(base + k * CHUNK, CHUNK)
            pltpu.sync_copy(acc.at[rows], buf)
            pltpu.sync_copy(buf, o_hbm.at[rows])
            return carry
        lax.fori_loop(0, RCHUNK, obody, 0)

    @pl.when(c == 0)
    def _():
        run_pass(t0, o0)
        run_pass(t1, o1)

    @pl.when(c == 1)
    def _():
        run_pass(t2, o2)
        run_pass(t3, o3)


def _final_body(o0_ref, o1_ref, o2_ref, o3_ref, dinv_ref, h3_ref, mask_ref,
                bg_ref, wl_ref, bl_ref, sc1_ref, sc2_ref):
    dinv = dinv_ref[...]
    wsum = jnp.sum(wl_ref[...], axis=1)
    bsum = jnp.sum(bl_ref[...])
    bg = bg_ref[...]
    acc1 = jnp.concatenate([o0_ref[...], o1_ref[...]], axis=1)
    h1 = jnp.maximum(acc1 * dinv[:, None] + bg[None, :], 0.0)
    acc2 = jnp.concatenate([o2_ref[...], o3_ref[...]], axis=1)
    h2 = jnp.maximum(acc2 * dinv[:, None] + bg[None, :], 0.0)
    h2 = jnp.where(mask_ref[...][:, None] > 0.0, h3_ref[...], h2)
    sc1_ref[...] = jnp.sum(h1 * wsum[None, :], axis=1) + bsum
    sc2_ref[...] = jnp.sum(h2 * wsum[None, :], axis=1) + bsum


_final_call = pl.pallas_call(
    _final_body,
    grid=(GRID,),
    in_specs=[
        pl.BlockSpec((BLK, 128), lambda i: (i, 0)),
        pl.BlockSpec((BLK, 128), lambda i: (i, 0)),
        pl.BlockSpec((BLK, 128), lambda i: (i, 0)),
        pl.BlockSpec((BLK, 128), lambda i: (i, 0)),
        pl.BlockSpec((BLK,), lambda i: (i,)),
        pl.BlockSpec((BLK, D), lambda i: (i, 0)),
        pl.BlockSpec((BLK,), lambda i: (i,)),
        pl.BlockSpec((D,), lambda i: (0,)),
        pl.BlockSpec((D, D), lambda i: (0, 0)),
        pl.BlockSpec((D,), lambda i: (0,)),
    ],
    out_specs=[
        pl.BlockSpec((BLK,), lambda i: (i,)),
        pl.BlockSpec((BLK,), lambda i: (i,)),
    ],
    out_shape=[jax.ShapeDtypeStruct((NP,), jnp.float32),
               jax.ShapeDtypeStruct((NP,), jnp.float32)],
)


def kernel(seq1, seq2, h_3, edge_index, W_gcn, b_gcn, W_lin, b_lin):
    src = edge_index[0]
    dst = edge_index[1]
    fill_d = jnp.full((E_PAD_D - E,), N, dtype=jnp.int32)
    dst_d = jnp.concatenate([dst, fill_d]).reshape(NC * NS, NCHUNK_D, CHUNK)

    deg0, deg1 = _deg_kernel(dst_d)

    seq1p = jnp.pad(seq1, ((0, NP - N), (0, 0)))
    seq2p = jnp.pad(seq2, ((0, NP - N), (0, 0)))
    t0, t1, t2, t3, dinvp = _dense_call(seq1p, seq2p, deg0, deg1, W_gcn)

    fill_a = jnp.full((E_PAD_A - E,), N, dtype=jnp.int32)
    src_a = jnp.concatenate([src, fill_a]).reshape(NS, NCHUNK, CHUNK)
    dst_a = jnp.concatenate([dst, fill_a]).reshape(NS, NCHUNK, CHUNK)
    o0, o1, o2, o3 = _agg_kernel(t0, t1, t2, t3, src_a, dst_a)

    s = jax.random.uniform(jax.random.key(42), (N,), dtype=jnp.float32)
    maskp = jnp.pad((s > 0.5).astype(jnp.float32), (0, NP - N))
    h3p = jnp.pad(h_3, ((0, NP - N), (0, 0)))
    sc1p, sc2p = _final_call(o0, o1, o2, o3, dinvp, h3p, maskp,
                             b_gcn, W_lin, b_lin)
    return jnp.concatenate([sc1p[:N], sc2p[:N]])
